# baseline (speedup 1.0000x reference)
import functools

import jax
import jax.numpy as jnp
from jax.experimental import pallas as pl
from jax.experimental.pallas import tpu as pltpu

F32 = jnp.float32


def _dot(a, b):
    return jax.lax.dot_general(a, b, (((1,), (0,)), ((), ())),
                               preferred_element_type=F32)


def _ln1_router_body(x_ref, g_ref, b_ref, rw_ref, rb_ref,
                     nx1_ref, probs_ref, acc_ref, *, nblocks, seq):
    i = pl.program_id(0)
    xb = x_ref[...]
    m = jnp.mean(xb, axis=-1, keepdims=True)
    v = jnp.mean(jnp.square(xb - m), axis=-1, keepdims=True)
    nx = (xb - m) / jnp.sqrt(v + 1e-6) * g_ref[...] + b_ref[...]
    nx1_ref[...] = nx
    part = _dot(jnp.sum(nx, axis=0, keepdims=True), rw_ref[...])

    @pl.when(i == 0)
    def _():
        acc_ref[...] = part

    @pl.when(i > 0)
    def _():
        acc_ref[...] += part

    @pl.when(i == nblocks - 1)
    def _():
        logits = acc_ref[...] / seq + rb_ref[...]
        mx = jnp.max(logits, axis=-1, keepdims=True)
        e = jnp.exp(logits - mx)
        probs_ref[...] = e / jnp.sum(e, axis=-1, keepdims=True)


def _mha_body(nx1_ref, wq_ref, wk_ref, wv_ref, bq_ref, bk_ref, bv_ref,
              o_ref, *, dh, q_blk, seq):
    scale = dh ** -0.5
    xb = nx1_ref[...]
    qh2 = _dot(xb, wq_ref[0]) + bq_ref[0]
    kh2 = _dot(xb, wk_ref[0]) + bk_ref[0]
    vh2 = _dot(xb, wv_ref[0]) + bv_ref[0]
    for hh in range(2):
        sl = slice(hh * dh, (hh + 1) * dh)
        kh = kh2[:, sl]
        vh = vh2[:, sl]
        qh = qh2[:, sl] * scale
        for qb in range(seq // q_blk):
            rows = slice(qb * q_blk, (qb + 1) * q_blk)
            s = jax.lax.dot_general(qh[rows], kh, (((1,), (1,)), ((), ())),
                                    preferred_element_type=F32)
            mx = jnp.max(s, axis=-1, keepdims=True)
            p = jnp.exp(s - mx)
            r = jnp.sum(p, axis=-1, keepdims=True)
            o_ref[0, rows, sl] = _dot(p, vh) * (1.0 / r)


def _outproj_ln2_body(x_ref, o_ref, wo_ref, bo_ref, probs_ref,
                      frw_ref, frb_ref, g2_ref, b2_ref,
                      x2_ref, nx2_ref, ti_ref, tp_ref, acc_ref,
                      *, num_e, num_s, seq, num_ffn):
    s = pl.program_id(0)
    e = pl.program_id(1)
    pe = probs_ref[0, e]
    contrib = (_dot(o_ref[0], wo_ref[0]) + bo_ref[0]) * pe

    @pl.when(e == 0)
    def _():
        x2_ref[...] = x_ref[...] + contrib

    @pl.when(e > 0)
    def _():
        x2_ref[...] += contrib

    @pl.when(e == num_e - 1)
    def _():
        x2 = x2_ref[...]
        m = jnp.mean(x2, axis=-1, keepdims=True)
        v = jnp.mean(jnp.square(x2 - m), axis=-1, keepdims=True)
        nx2 = (x2 - m) / jnp.sqrt(v + 1e-6) * g2_ref[...] + b2_ref[...]
        nx2_ref[...] = nx2
        part = _dot(jnp.sum(nx2, axis=0, keepdims=True), frw_ref[...])

        @pl.when(s == 0)
        def _():
            acc_ref[...] = part

        @pl.when(s > 0)
        def _():
            acc_ref[...] += part

        @pl.when(s == num_s - 1)
        def _():
            logits = acc_ref[...] / seq + frb_ref[...]
            mx = jnp.max(logits, axis=-1, keepdims=True)
            ex = jnp.exp(logits - mx)
            p8 = ex / jnp.sum(ex, axis=-1, keepdims=True)
            iota = jax.lax.broadcasted_iota(jnp.int32, p8.shape, 1)
            big = jnp.int32(10 ** 9)
            v1 = jnp.max(p8)
            i1 = jnp.min(jnp.where(p8 == v1, iota, big))
            pm = jnp.where(iota == i1, jnp.float32(-2.0), p8)
            v2 = jnp.max(pm)
            i2 = jnp.min(jnp.where(pm == v2, iota, big))
            tot = v1 + v2
            ti_ref[0] = i1
            ti_ref[1] = i2
            tp_ref[0] = v1 / tot
            tp_ref[1] = v2 / tot


def _ffn_body(ti_ref, nx2_ref, w1_ref, w2_ref, tp_ref, x2_ref, out_ref):
    k = pl.program_id(1)
    f = pl.program_id(2)

    @pl.when((k == 0) & (f == 0))
    def _():
        out_ref[...] = x2_ref[...]

    h = _dot(nx2_ref[...], w1_ref[0])
    h = 0.5 * h * (1.0 + jax.lax.erf(h * (2.0 ** -0.5)))
    out_ref[...] += tp_ref[k] * _dot(h, w2_ref[0])


def kernel(x, ln1_g, ln1_b, mha_Wq, mha_bq, mha_Wk, mha_bk, mha_Wv, mha_bv,
           mha_Wo, mha_bo, mha_rW, mha_rb, ln2_g, ln2_b, ffn_rW, ffn_rb,
           fw1, fw2):
    B, S, D = x.shape
    E, _, H, DH = mha_Wq.shape
    NF = fw1.shape[0]
    F2 = fw1.shape[2]
    HD = H * DH
    x2d = x.reshape(S, D)

    SB = 256
    nb = S // SB
    nx1, mha_probs = pl.pallas_call(
        functools.partial(_ln1_router_body, nblocks=nb, seq=S),
        grid=(nb,),
        in_specs=[
            pl.BlockSpec((SB, D), lambda i: (i, 0)),
            pl.BlockSpec((1, D), lambda i: (0, 0)),
            pl.BlockSpec((1, D), lambda i: (0, 0)),
            pl.BlockSpec((D, E), lambda i: (0, 0)),
            pl.BlockSpec((1, E), lambda i: (0, 0)),
        ],
        out_specs=[
            pl.BlockSpec((SB, D), lambda i: (i, 0)),
            pl.BlockSpec((1, E), lambda i: (0, 0)),
        ],
        out_shape=[
            jax.ShapeDtypeStruct((S, D), F32),
            jax.ShapeDtypeStruct((1, E), F32),
        ],
        scratch_shapes=[pltpu.VMEM((1, E), F32)],
    )(x2d, ln1_g.reshape(1, D), ln1_b.reshape(1, D), mha_rW,
      mha_rb.reshape(1, E))

    QBLK = 256
    HB = 2 * DH
    o_all = pl.pallas_call(
        functools.partial(_mha_body, dh=DH, q_blk=QBLK, seq=S),
        grid=(E, H // 2),
        in_specs=[
            pl.BlockSpec((S, D), lambda e, h: (0, 0)),
            pl.BlockSpec((1, D, HB), lambda e, h: (e, 0, h)),
            pl.BlockSpec((1, D, HB), lambda e, h: (e, 0, h)),
            pl.BlockSpec((1, D, HB), lambda e, h: (e, 0, h)),
            pl.BlockSpec((1, 1, HB), lambda e, h: (e, 0, h)),
            pl.BlockSpec((1, 1, HB), lambda e, h: (e, 0, h)),
            pl.BlockSpec((1, 1, HB), lambda e, h: (e, 0, h)),
        ],
        out_specs=pl.BlockSpec((1, S, HB), lambda e, h: (e, 0, h)),
        out_shape=jax.ShapeDtypeStruct((E, S, HD), F32),
        compiler_params=pltpu.CompilerParams(
            dimension_semantics=("parallel", "parallel")),
    )(nx1, mha_Wq.reshape(E, D, HD), mha_Wk.reshape(E, D, HD),
      mha_Wv.reshape(E, D, HD), mha_bq.reshape(E, 1, HD),
      mha_bk.reshape(E, 1, HD), mha_bv.reshape(E, 1, HD))

    SB3 = 1024
    ns3 = S // SB3
    x2, nx2, ti, tp = pl.pallas_call(
        functools.partial(_outproj_ln2_body, num_e=E, num_s=ns3, seq=S,
                          num_ffn=NF),
        grid=(ns3, E),
        in_specs=[
            pl.BlockSpec((SB3, D), lambda s, e: (s, 0)),
            pl.BlockSpec((1, SB3, HD), lambda s, e: (e, s, 0)),
            pl.BlockSpec((1, HD, D), lambda s, e: (e, 0, 0)),
            pl.BlockSpec((1, 1, D), lambda s, e: (e, 0, 0)),
            pl.BlockSpec(memory_space=pltpu.SMEM),
            pl.BlockSpec((D, NF), lambda s, e: (0, 0)),
            pl.BlockSpec((1, NF), lambda s, e: (0, 0)),
            pl.BlockSpec((1, D), lambda s, e: (0, 0)),
            pl.BlockSpec((1, D), lambda s, e: (0, 0)),
        ],
        out_specs=[
            pl.BlockSpec((SB3, D), lambda s, e: (s, 0)),
            pl.BlockSpec((SB3, D), lambda s, e: (s, 0)),
            pl.BlockSpec(memory_space=pltpu.SMEM),
            pl.BlockSpec(memory_space=pltpu.SMEM),
        ],
        out_shape=[
            jax.ShapeDtypeStruct((S, D), F32),
            jax.ShapeDtypeStruct((S, D), F32),
            jax.ShapeDtypeStruct((2,), jnp.int32),
            jax.ShapeDtypeStruct((2,), F32),
        ],
        scratch_shapes=[pltpu.VMEM((1, NF), F32)],
    )(x2d, o_all, mha_Wo.reshape(E, HD, D), mha_bo.reshape(E, 1, D), mha_probs,
      ffn_rW, ffn_rb.reshape(1, NF), ln2_g.reshape(1, D), ln2_b.reshape(1, D))

    SBF = 1024
    FBLK = 512
    grid_spec = pltpu.PrefetchScalarGridSpec(
        num_scalar_prefetch=1,
        grid=(S // SBF, 2, F2 // FBLK),
        in_specs=[
            pl.BlockSpec((SBF, D), lambda s, kk, f, ti: (s, 0)),
            pl.BlockSpec((1, D, FBLK), lambda s, kk, f, ti: (ti[kk], 0, f)),
            pl.BlockSpec((1, FBLK, D), lambda s, kk, f, ti: (ti[kk], f, 0)),
            pl.BlockSpec(memory_space=pltpu.SMEM),
            pl.BlockSpec((SBF, D), lambda s, kk, f, ti: (s, 0)),
        ],
        out_specs=pl.BlockSpec((SBF, D), lambda s, kk, f, ti: (s, 0)),
    )
    out2d = pl.pallas_call(
        _ffn_body,
        grid_spec=grid_spec,
        out_shape=jax.ShapeDtypeStruct((S, D), F32),
        compiler_params=pltpu.CompilerParams(
            dimension_semantics=("parallel", "arbitrary", "arbitrary")),
    )(ti, nx2, fw1, fw2, tp, x2)

    return out2d.reshape(B, S, D)


# speedup vs baseline: 1.0976x; 1.0976x over previous
import functools

import jax
import jax.numpy as jnp
from jax.experimental import pallas as pl
from jax.experimental.pallas import tpu as pltpu

F32 = jnp.float32


def _dot(a, b):
    return jax.lax.dot_general(a, b, (((1,), (0,)), ((), ())),
                               preferred_element_type=F32)


def _ln1_router_body(x_ref, g_ref, b_ref, rw_ref, rb_ref,
                     nx1_ref, probs_ref, acc_ref, *, nblocks, seq):
    i = pl.program_id(0)
    xb = x_ref[...]
    m = jnp.mean(xb, axis=-1, keepdims=True)
    v = jnp.mean(jnp.square(xb - m), axis=-1, keepdims=True)
    nx = (xb - m) / jnp.sqrt(v + 1e-6) * g_ref[...] + b_ref[...]
    nx1_ref[...] = nx
    part = _dot(jnp.sum(nx, axis=0, keepdims=True), rw_ref[...])

    @pl.when(i == 0)
    def _():
        acc_ref[...] = part

    @pl.when(i > 0)
    def _():
        acc_ref[...] += part

    @pl.when(i == nblocks - 1)
    def _():
        logits = acc_ref[...] / seq + rb_ref[...]
        mx = jnp.max(logits, axis=-1, keepdims=True)
        e = jnp.exp(logits - mx)
        probs_ref[...] = e / jnp.sum(e, axis=-1, keepdims=True)


def _qkv_body(nx1_ref, wq_ref, wk_ref, wv_ref, bq_ref, bk_ref, bv_ref,
              q_ref, k_ref, v_ref):
    xb = nx1_ref[...]
    q_ref[0] = _dot(xb, wq_ref[0]) + bq_ref[0]
    k_ref[0] = _dot(xb, wk_ref[0]) + bk_ref[0]
    v_ref[0] = _dot(xb, wv_ref[0]) + bv_ref[0]


def _attn_body(q_ref, k_ref, v_ref, o_ref, *, dh, q_blk, seq):
    scale = dh ** -0.5
    for hh in range(2):
        sl = slice(hh * dh, (hh + 1) * dh)
        kh = k_ref[0, :, sl]
        vh = v_ref[0, :, sl]
        for qb in range(seq // q_blk):
            rows = slice(qb * q_blk, (qb + 1) * q_blk)
            qblk = q_ref[0, rows, sl] * scale
            s = jax.lax.dot_general(qblk, kh, (((1,), (1,)), ((), ())),
                                    preferred_element_type=F32)
            mx = jnp.max(s, axis=-1, keepdims=True)
            p = jnp.exp(s - mx)
            r = jnp.sum(p, axis=-1, keepdims=True)
            o_ref[0, rows, sl] = _dot(p, vh) * (1.0 / r)


def _outproj_ln2_body(x_ref, o_ref, wo_ref, bo_ref, probs_ref,
                      frw_ref, frb_ref, g2_ref, b2_ref,
                      x2_ref, nx2_ref, ti_ref, tp_ref, acc_ref,
                      *, num_e, num_s, seq, num_ffn):
    s = pl.program_id(0)
    e = pl.program_id(1)
    pe = probs_ref[0, e]
    contrib = (_dot(o_ref[0], wo_ref[0]) + bo_ref[0]) * pe

    @pl.when(e == 0)
    def _():
        x2_ref[...] = x_ref[...] + contrib

    @pl.when(e > 0)
    def _():
        x2_ref[...] += contrib

    @pl.when(e == num_e - 1)
    def _():
        x2 = x2_ref[...]
        m = jnp.mean(x2, axis=-1, keepdims=True)
        v = jnp.mean(jnp.square(x2 - m), axis=-1, keepdims=True)
        nx2 = (x2 - m) / jnp.sqrt(v + 1e-6) * g2_ref[...] + b2_ref[...]
        nx2_ref[...] = nx2
        part = _dot(jnp.sum(nx2, axis=0, keepdims=True), frw_ref[...])

        @pl.when(s == 0)
        def _():
            acc_ref[...] = part

        @pl.when(s > 0)
        def _():
            acc_ref[...] += part

        @pl.when(s == num_s - 1)
        def _():
            logits = acc_ref[...] / seq + frb_ref[...]
            mx = jnp.max(logits, axis=-1, keepdims=True)
            ex = jnp.exp(logits - mx)
            p8 = ex / jnp.sum(ex, axis=-1, keepdims=True)
            iota = jax.lax.broadcasted_iota(jnp.int32, p8.shape, 1)
            big = jnp.int32(10 ** 9)
            v1 = jnp.max(p8)
            i1 = jnp.min(jnp.where(p8 == v1, iota, big))
            pm = jnp.where(iota == i1, jnp.float32(-2.0), p8)
            v2 = jnp.max(pm)
            i2 = jnp.min(jnp.where(pm == v2, iota, big))
            tot = v1 + v2
            ti_ref[0] = i1
            ti_ref[1] = i2
            tp_ref[0] = v1 / tot
            tp_ref[1] = v2 / tot


def _ffn_body(ti_ref, nx2_ref, w1_ref, w2_ref, tp_ref, x2_ref, out_ref):
    k = pl.program_id(1)
    f = pl.program_id(2)

    @pl.when((k == 0) & (f == 0))
    def _():
        out_ref[...] = x2_ref[...]

    h = _dot(nx2_ref[...], w1_ref[0])
    h = 0.5 * h * (1.0 + jax.lax.erf(h * (2.0 ** -0.5)))
    out_ref[...] += tp_ref[k] * _dot(h, w2_ref[0])


def kernel(x, ln1_g, ln1_b, mha_Wq, mha_bq, mha_Wk, mha_bk, mha_Wv, mha_bv,
           mha_Wo, mha_bo, mha_rW, mha_rb, ln2_g, ln2_b, ffn_rW, ffn_rb,
           fw1, fw2):
    B, S, D = x.shape
    E, _, H, DH = mha_Wq.shape
    NF = fw1.shape[0]
    F2 = fw1.shape[2]
    HD = H * DH
    x2d = x.reshape(S, D)

    SB = 256
    nb = S // SB
    nx1, mha_probs = pl.pallas_call(
        functools.partial(_ln1_router_body, nblocks=nb, seq=S),
        grid=(nb,),
        in_specs=[
            pl.BlockSpec((SB, D), lambda i: (i, 0)),
            pl.BlockSpec((1, D), lambda i: (0, 0)),
            pl.BlockSpec((1, D), lambda i: (0, 0)),
            pl.BlockSpec((D, E), lambda i: (0, 0)),
            pl.BlockSpec((1, E), lambda i: (0, 0)),
        ],
        out_specs=[
            pl.BlockSpec((SB, D), lambda i: (i, 0)),
            pl.BlockSpec((1, E), lambda i: (0, 0)),
        ],
        out_shape=[
            jax.ShapeDtypeStruct((S, D), F32),
            jax.ShapeDtypeStruct((1, E), F32),
        ],
        scratch_shapes=[pltpu.VMEM((1, E), F32)],
    )(x2d, ln1_g.reshape(1, D), ln1_b.reshape(1, D), mha_rW,
      mha_rb.reshape(1, E))

    NBLK = 256
    nn = HD // NBLK
    qkv_shape = jax.ShapeDtypeStruct((E, S, HD), F32)
    q, k, v = pl.pallas_call(
        _qkv_body,
        grid=(E, nn),
        in_specs=[
            pl.BlockSpec((S, D), lambda e, n: (0, 0)),
            pl.BlockSpec((1, D, NBLK), lambda e, n: (e, 0, n)),
            pl.BlockSpec((1, D, NBLK), lambda e, n: (e, 0, n)),
            pl.BlockSpec((1, D, NBLK), lambda e, n: (e, 0, n)),
            pl.BlockSpec((1, 1, NBLK), lambda e, n: (e, 0, n)),
            pl.BlockSpec((1, 1, NBLK), lambda e, n: (e, 0, n)),
            pl.BlockSpec((1, 1, NBLK), lambda e, n: (e, 0, n)),
        ],
        out_specs=[
            pl.BlockSpec((1, S, NBLK), lambda e, n: (e, 0, n)),
            pl.BlockSpec((1, S, NBLK), lambda e, n: (e, 0, n)),
            pl.BlockSpec((1, S, NBLK), lambda e, n: (e, 0, n)),
        ],
        out_shape=[qkv_shape, qkv_shape, qkv_shape],
        compiler_params=pltpu.CompilerParams(
            dimension_semantics=("parallel", "parallel")),
    )(nx1, mha_Wq.reshape(E, D, HD), mha_Wk.reshape(E, D, HD),
      mha_Wv.reshape(E, D, HD), mha_bq.reshape(E, 1, HD),
      mha_bk.reshape(E, 1, HD), mha_bv.reshape(E, 1, HD))

    QBLK = 256
    o_all = pl.pallas_call(
        functools.partial(_attn_body, dh=DH, q_blk=QBLK, seq=S),
        grid=(E, H // 2),
        in_specs=[
            pl.BlockSpec((1, S, 2 * DH), lambda e, h: (e, 0, h)),
            pl.BlockSpec((1, S, 2 * DH), lambda e, h: (e, 0, h)),
            pl.BlockSpec((1, S, 2 * DH), lambda e, h: (e, 0, h)),
        ],
        out_specs=pl.BlockSpec((1, S, 2 * DH), lambda e, h: (e, 0, h)),
        out_shape=jax.ShapeDtypeStruct((E, S, HD), F32),
        compiler_params=pltpu.CompilerParams(
            dimension_semantics=("parallel", "parallel")),
    )(q, k, v)

    SB3 = 1024
    ns3 = S // SB3
    x2, nx2, ti, tp = pl.pallas_call(
        functools.partial(_outproj_ln2_body, num_e=E, num_s=ns3, seq=S,
                          num_ffn=NF),
        grid=(ns3, E),
        in_specs=[
            pl.BlockSpec((SB3, D), lambda s, e: (s, 0)),
            pl.BlockSpec((1, SB3, HD), lambda s, e: (e, s, 0)),
            pl.BlockSpec((1, HD, D), lambda s, e: (e, 0, 0)),
            pl.BlockSpec((1, 1, D), lambda s, e: (e, 0, 0)),
            pl.BlockSpec(memory_space=pltpu.SMEM),
            pl.BlockSpec((D, NF), lambda s, e: (0, 0)),
            pl.BlockSpec((1, NF), lambda s, e: (0, 0)),
            pl.BlockSpec((1, D), lambda s, e: (0, 0)),
            pl.BlockSpec((1, D), lambda s, e: (0, 0)),
        ],
        out_specs=[
            pl.BlockSpec((SB3, D), lambda s, e: (s, 0)),
            pl.BlockSpec((SB3, D), lambda s, e: (s, 0)),
            pl.BlockSpec(memory_space=pltpu.SMEM),
            pl.BlockSpec(memory_space=pltpu.SMEM),
        ],
        out_shape=[
            jax.ShapeDtypeStruct((S, D), F32),
            jax.ShapeDtypeStruct((S, D), F32),
            jax.ShapeDtypeStruct((2,), jnp.int32),
            jax.ShapeDtypeStruct((2,), F32),
        ],
        scratch_shapes=[pltpu.VMEM((1, NF), F32)],
    )(x2d, o_all, mha_Wo.reshape(E, HD, D), mha_bo.reshape(E, 1, D), mha_probs,
      ffn_rW, ffn_rb.reshape(1, NF), ln2_g.reshape(1, D), ln2_b.reshape(1, D))

    SBF = 1024
    FBLK = 512
    grid_spec = pltpu.PrefetchScalarGridSpec(
        num_scalar_prefetch=1,
        grid=(S // SBF, 2, F2 // FBLK),
        in_specs=[
            pl.BlockSpec((SBF, D), lambda s, kk, f, ti: (s, 0)),
            pl.BlockSpec((1, D, FBLK), lambda s, kk, f, ti: (ti[kk], 0, f)),
            pl.BlockSpec((1, FBLK, D), lambda s, kk, f, ti: (ti[kk], f, 0)),
            pl.BlockSpec(memory_space=pltpu.SMEM),
            pl.BlockSpec((SBF, D), lambda s, kk, f, ti: (s, 0)),
        ],
        out_specs=pl.BlockSpec((SBF, D), lambda s, kk, f, ti: (s, 0)),
    )
    out2d = pl.pallas_call(
        _ffn_body,
        grid_spec=grid_spec,
        out_shape=jax.ShapeDtypeStruct((S, D), F32),
        compiler_params=pltpu.CompilerParams(
            dimension_semantics=("parallel", "arbitrary", "arbitrary")),
    )(ti, nx2, fw1, fw2, tp, x2)

    return out2d.reshape(B, S, D)
